# baseline (speedup 1.0000x reference)
import functools

import jax
import jax.numpy as jnp
from jax import lax
from jax.experimental import pallas as pl
from jax.experimental.pallas import tpu as pltpu
from jax.experimental.pallas import tpu_sc as plsc

AN = 5
D_MODEL = 64
NHEAD = 8
DPH = D_MODEL // NHEAD
DFF = 256
N = 10000
NB = 16
DA = D_MODEL * AN
DP = 384

NCHUNK = 5
NC = N // NCHUNK
PAIRS_C = NC * NB

NWORK = 32
PER_W = PAIRS_C // NWORK
WIN = 144
NWIN = -(-PER_W // WIN)
LAST_OFF = PER_W - WIN

BA = 1000
BC = 400

_MASK_HI = -65536


def _to_bf16_bits(x):
    b = jax.lax.bitcast_convert_type(x, jnp.int32)
    return ((b + 0x7FFF + ((b >> 16) & 1)) >> 16) & 0xFFFF


def _proj_body(tgt_ref, wk_ref, wq_ref, kv_ref):
    t = tgt_ref[...]
    kf = jnp.dot(t, wk_ref[...], preferred_element_type=jnp.float32)
    vf = jnp.dot(t, wq_ref[...], preferred_element_type=jnp.float32)
    kv_ref[...] = (_to_bf16_bits(vf) << 16) | _to_bf16_bits(kf)


def _project(tgt, wkp, wqp):
    grid = (N // BA,)
    return pl.pallas_call(
        _proj_body,
        grid=grid,
        in_specs=[
            pl.BlockSpec((BA, DA), lambda i: (i, 0)),
            pl.BlockSpec((DA, DP), lambda i: (0, 0)),
            pl.BlockSpec((DA, DP), lambda i: (0, 0)),
        ],
        out_specs=pl.BlockSpec((BA, DP), lambda i: (i, 0)),
        out_shape=jax.ShapeDtypeStruct((N, DP), jnp.int32),
    )(tgt, wkp, wqp)


def _gather(kv, idx_t, c0):
    mesh = plsc.VectorSubcoreMesh(core_axis_name="c", subcore_axis_name="s")

    @functools.partial(
        pl.kernel,
        out_type=jax.ShapeDtypeStruct((PAIRS_C, DP), jnp.int32),
        mesh=mesh,
        scratch_types=[
            pltpu.VMEM((PER_W,), jnp.int32),
            pltpu.VMEM((WIN, DP), jnp.int32),
            pltpu.VMEM((WIN, DP), jnp.int32),
            pltpu.SemaphoreType.DMA,
            pltpu.SemaphoreType.DMA,
            pltpu.SemaphoreType.DMA,
            pltpu.SemaphoreType.DMA,
        ],
    )
    def k(kv_hbm, idx_hbm, g_hbm, idx_v, b0, b1, sg0, sg1, sw0, sw1):
        wid = lax.axis_index("s") * 2 + lax.axis_index("c")
        base = pl.multiple_of(wid * PER_W, 8)
        pltpu.sync_copy(
            idx_hbm.at[pl.ds(c0 * PAIRS_C + base, PER_W)], idx_v)

        bufs = (b0, b1)
        gsem = (sg0, sg1)
        wsem = (sw0, sw1)

        def wait_gather(par):
            pltpu.make_async_copy(
                kv_hbm.at[pl.ds(0, WIN)], bufs[par], gsem[par]).wait()

        def wait_write(par):
            pltpu.make_async_copy(
                bufs[par], g_hbm.at[pl.ds(0, WIN)], wsem[par]).wait()

        pltpu.async_copy(kv_hbm.at[idx_v.at[pl.ds(0, WIN)]], b0, sg0)

        @pl.loop(0, NWIN)
        def _(w):
            for par in (0, 1):
                oth = 1 - par

                @pl.when(lax.rem(w, 2) == par)
                def _(par=par, oth=oth):
                    @pl.when(w >= 1)
                    def _():
                        wait_write(oth)

                    @pl.when(w + 1 < NWIN)
                    def _():
                        off_n = pl.multiple_of(
                            jnp.minimum(w * WIN + WIN, LAST_OFF), 8)
                        pltpu.async_copy(
                            kv_hbm.at[idx_v.at[pl.ds(off_n, WIN)]],
                            bufs[oth], gsem[oth])

                    wait_gather(par)
                    off = pl.multiple_of(jnp.minimum(w * WIN, LAST_OFF), 8)
                    pltpu.async_copy(
                        bufs[par], g_hbm.at[pl.ds(base + off, WIN)],
                        wsem[par])

        wait_write((NWIN - 1) % 2)

    return k(kv, idx_t)


def _attn_body(tgt_ref, own_ref, g_ref, dist_ref, p_ref, h8_ref,
               h8t_ref, wo_ref, ma_ref, mb5_ref, w1_ref, b1_ref, w2_ref,
               b2_ref, g1_ref, be1_ref, g2p_ref, be2d_ref, out_ref):
    scale = 1.0 / (DPH ** 0.5)
    qv = jax.lax.bitcast_convert_type(own_ref[...] & _MASK_HI, jnp.float32)
    h8 = h8_ref[...]
    gs = [g_ref[n] for n in range(NB)]
    logits = []
    for n in range(NB):
        kf = jax.lax.bitcast_convert_type(gs[n] << 16, jnp.float32)
        dn = jnp.dot(qv * kf, h8, preferred_element_type=jnp.float32)
        logits.append((dn + dist_ref[:, n, :]) * scale)
    m = logits[0]
    for n in range(1, NB):
        m = jnp.maximum(m, logits[n])
    es = [jnp.exp(l - m) for l in logits]
    s = es[0]
    for n in range(1, NB):
        s = s + es[n]
    inv = 1.0 / s
    h8t = h8t_ref[...]
    acc = jnp.zeros((BC, DP), jnp.float32)
    for n in range(NB):
        w = es[n] * inv
        vf = jax.lax.bitcast_convert_type(gs[n] & _MASK_HI, jnp.float32)
        acc = acc + (jnp.dot(w, h8t, preferred_element_type=jnp.float32)
                     * vf)

    tp = jnp.dot(tgt_ref[...], p_ref[...], preferred_element_type=jnp.float32)
    x = tp + jnp.dot(acc, wo_ref[...], preferred_element_type=jnp.float32)
    mm5 = ma_ref[...]
    mb5 = mb5_ref[...]
    mb = jnp.dot(jnp.dot(x, mm5, preferred_element_type=jnp.float32), mb5,
                 preferred_element_type=jnp.float32)
    xc = x - mb
    vb = jnp.dot(jnp.dot(xc * xc, mm5, preferred_element_type=jnp.float32),
                 mb5, preferred_element_type=jnp.float32)
    xn = xc * lax.rsqrt(vb + 1e-5) * g1_ref[...] + be1_ref[...]
    h = jnp.maximum(
        jnp.dot(xn.astype(jnp.bfloat16), w1_ref[...],
                preferred_element_type=jnp.float32)
        + b1_ref[...], 0.0)
    x2 = xn + (jnp.dot(h.astype(jnp.bfloat16), w2_ref[...],
                       preferred_element_type=jnp.float32)
               + b2_ref[...])
    mb2 = jnp.dot(jnp.dot(x2, mm5, preferred_element_type=jnp.float32), mb5,
                  preferred_element_type=jnp.float32)
    xc2 = x2 - mb2
    vb2 = jnp.dot(jnp.dot(xc2 * xc2, mm5, preferred_element_type=jnp.float32),
                  mb5, preferred_element_type=jnp.float32)
    out_ref[...] = (jnp.dot(xc2 * lax.rsqrt(vb2 + 1e-5), g2p_ref[...],
                            preferred_element_type=jnp.float32)
                    + be2d_ref[...])


def _attn_tail(c0, tgt, kvp, g3, dist_t, p_mat, h8, h8t, wo, mm5, mb5,
               w1k, b1k, w2k, b2k, g1t, be1t, g2p, be2d):
    grid = (NC // BC,)
    nblk = NC // BC
    full = lambda r, c: pl.BlockSpec((r, c), lambda i: (0, 0))
    return pl.pallas_call(
        _attn_body,
        grid=grid,
        in_specs=[
            pl.BlockSpec((BC, DA), lambda i: (c0 * nblk + i, 0)),
            pl.BlockSpec((BC, DP), lambda i: (c0 * nblk + i, 0)),
            pl.BlockSpec((NB, BC, DP), lambda i: (0, i, 0)),
            pl.BlockSpec((BC, NB, NHEAD),
                         lambda i: (c0 * nblk + i, 0, 0)),
            full(DA, DA),
            full(DP, NHEAD),
            full(NHEAD, DP),
            full(DP, DA),
            full(DA, AN),
            full(AN, DA),
            full(DA, AN * DFF),
            full(1, AN * DFF),
            full(AN * DFF, DA),
            full(1, DA),
            full(1, DA),
            full(1, DA),
            full(DA, DA),
            full(1, DA),
        ],
        out_specs=pl.BlockSpec((BC, DA), lambda i: (i, 0)),
        out_shape=jax.ShapeDtypeStruct((NC, DA), jnp.float32),
    )(tgt, kvp, g3, dist_t, p_mat, h8, h8t, wo, mm5, mb5, w1k, b1k, w2k,
      b2k, g1t, be1t, g2p, be2d)


def kernel(tgt, index_pair, cnt, sh, dist_atten, Wq, Wk, Wout, W1, b1, W2,
           b2, g1, be1, g2, be2):
    del cnt, sh
    f32 = jnp.float32
    eye5 = jnp.eye(AN, dtype=f32)

    def expand_in(w):
        m = w.T[:, None, None, :] * eye5[None, :, :, None]
        return m.reshape(DA, DA)

    pad = jnp.zeros((DA, DP - DA), f32)
    wkp = jnp.concatenate([expand_in(Wk), pad], axis=1)
    wqp = jnp.concatenate([expand_in(Wq), pad], axis=1)
    p_mat = expand_in(jnp.eye(D_MODEL, dtype=f32))
    wo = jnp.kron(eye5, Wout.T)
    wo384 = jnp.concatenate([wo, jnp.zeros((DP - DA, DA), f32)], axis=0)
    mm5 = jnp.kron(eye5, jnp.full((D_MODEL, 1), 1.0 / D_MODEL, f32))
    mb5 = jnp.kron(eye5, jnp.ones((1, D_MODEL), f32))
    w1k = jnp.kron(eye5, W1.T).astype(jnp.bfloat16)
    w2k = jnp.kron(eye5, W2.T).astype(jnp.bfloat16)
    b1k = jnp.tile(b1, AN)[None, :]
    b2k = jnp.tile(b2, AN)[None, :]
    g1t = jnp.tile(g1, AN)[None, :]
    be1t = jnp.tile(be1, AN)[None, :]
    g2t = jnp.tile(g2, AN)
    be2t = jnp.tile(be2, AN)
    dmaj = jnp.tile(jnp.arange(D_MODEL), AN)
    h8 = (dmaj[:, None] // DPH == jnp.arange(NHEAD)[None, :]).astype(f32)
    h8pad = jnp.concatenate([h8, jnp.zeros((DP - DA, NHEAD), f32)], axis=0)
    h8t = jnp.concatenate([h8.T, jnp.zeros((NHEAD, DP - DA), f32)], axis=1)
    g2p = g2t[:, None] * p_mat.T
    be2d = be2t[None, :] @ p_mat.T

    kvp = _project(tgt, wkp, wqp)
    idx_full = jnp.transpose(
        index_pair.astype(jnp.int32).reshape(NCHUNK, NC, NB),
        (0, 2, 1)).reshape(-1)
    outs = []
    for c in range(NCHUNK):
        g3 = _gather(kvp, idx_full, c).reshape(NB, NC, DP)
        outs.append(_attn_tail(c, tgt, kvp, g3, dist_atten, p_mat, h8pad,
                               h8t, wo384, mm5, mb5, w1k, b1k, w2k, b2k,
                               g1t, be1t, g2p, be2d))
    return jnp.concatenate(outs, axis=0)


# speedup vs baseline: 1.0474x; 1.0474x over previous
import functools

import jax
import jax.numpy as jnp
from jax import lax
from jax.experimental import pallas as pl
from jax.experimental.pallas import tpu as pltpu
from jax.experimental.pallas import tpu_sc as plsc

AN = 5
D_MODEL = 64
NHEAD = 8
DPH = D_MODEL // NHEAD
DFF = 256
N = 10000
NB = 16
DA = D_MODEL * AN
DP = 384

NCHUNK = 5
NC = N // NCHUNK
PAIRS_C = NC * NB

NWORK = 32
PER_W = PAIRS_C // NWORK
WIN = 144
NWIN = -(-PER_W // WIN)
LAST_OFF = PER_W - WIN

BA = 1000
BC = 400

_MASK_HI = -65536


def _to_bf16_bits(x):
    b = jax.lax.bitcast_convert_type(x, jnp.int32)
    return ((b + 0x7FFF + ((b >> 16) & 1)) >> 16) & 0xFFFF


def _proj_body(tgt_ref, wk_ref, wq_ref, kv_ref):
    t = tgt_ref[...]
    kf = jnp.dot(t, wk_ref[...], preferred_element_type=jnp.float32)
    vf = jnp.dot(t, wq_ref[...], preferred_element_type=jnp.float32)
    kv_ref[...] = (_to_bf16_bits(vf) << 16) | _to_bf16_bits(kf)


def _project(tgt, wkp, wqp):
    grid = (N // BA,)
    return pl.pallas_call(
        _proj_body,
        grid=grid,
        in_specs=[
            pl.BlockSpec((BA, DA), lambda i: (i, 0)),
            pl.BlockSpec((DA, DP), lambda i: (0, 0)),
            pl.BlockSpec((DA, DP), lambda i: (0, 0)),
        ],
        out_specs=pl.BlockSpec((BA, DP), lambda i: (i, 0)),
        out_shape=jax.ShapeDtypeStruct((N, DP), jnp.int32),
    )(tgt, wkp, wqp)


def _gather(kv, idx_t, c0):
    mesh = plsc.VectorSubcoreMesh(core_axis_name="c", subcore_axis_name="s")

    @functools.partial(
        pl.kernel,
        out_type=jax.ShapeDtypeStruct((PAIRS_C, DP), jnp.int32),
        mesh=mesh,
        scratch_types=[
            pltpu.VMEM((PER_W,), jnp.int32),
            pltpu.VMEM((WIN, DP), jnp.int32),
            pltpu.VMEM((WIN, DP), jnp.int32),
            pltpu.SemaphoreType.DMA,
            pltpu.SemaphoreType.DMA,
            pltpu.SemaphoreType.DMA,
            pltpu.SemaphoreType.DMA,
        ],
    )
    def k(kv_hbm, idx_hbm, g_hbm, idx_v, b0, b1, sg0, sg1, sw0, sw1):
        wid = lax.axis_index("s") * 2 + lax.axis_index("c")
        base = pl.multiple_of(wid * PER_W, 8)
        pltpu.sync_copy(
            idx_hbm.at[pl.ds(c0 * PAIRS_C + base, PER_W)], idx_v)

        bufs = (b0, b1)
        gsem = (sg0, sg1)
        wsem = (sw0, sw1)

        def wait_gather(par):
            pltpu.make_async_copy(
                kv_hbm.at[pl.ds(0, WIN)], bufs[par], gsem[par]).wait()

        def wait_write(par):
            pltpu.make_async_copy(
                bufs[par], g_hbm.at[pl.ds(0, WIN)], wsem[par]).wait()

        pltpu.async_copy(kv_hbm.at[idx_v.at[pl.ds(0, WIN)]], b0, sg0)

        @pl.loop(0, NWIN)
        def _(w):
            for par in (0, 1):
                oth = 1 - par

                @pl.when(lax.rem(w, 2) == par)
                def _(par=par, oth=oth):
                    @pl.when(w >= 1)
                    def _():
                        wait_write(oth)

                    @pl.when(w + 1 < NWIN)
                    def _():
                        off_n = pl.multiple_of(
                            jnp.minimum(w * WIN + WIN, LAST_OFF), 8)
                        pltpu.async_copy(
                            kv_hbm.at[idx_v.at[pl.ds(off_n, WIN)]],
                            bufs[oth], gsem[oth])

                    wait_gather(par)
                    off = pl.multiple_of(jnp.minimum(w * WIN, LAST_OFF), 8)
                    pltpu.async_copy(
                        bufs[par], g_hbm.at[pl.ds(base + off, WIN)],
                        wsem[par])

        wait_write((NWIN - 1) % 2)

    return k(kv, idx_t)


def _attn_body(tgt_ref, own_ref, g_ref, dist_ref, p_ref, h8_ref,
               h8t_ref, wo_ref, ma_ref, mb5_ref, w1_ref, b1_ref, w2_ref,
               b2_ref, g1_ref, be1_ref, g2p_ref, be2d_ref, out_ref):
    scale = 1.0 / (DPH ** 0.5)
    qv = jax.lax.bitcast_convert_type(own_ref[...] & _MASK_HI, jnp.float32)
    h8 = h8_ref[...]
    gs = [g_ref[n] for n in range(NB)]
    logits = []
    for n in range(NB):
        kf = jax.lax.bitcast_convert_type(gs[n] << 16, jnp.float32)
        dn = jnp.dot(qv * kf, h8, preferred_element_type=jnp.float32)
        logits.append((dn + dist_ref[n]) * scale)
    m = logits[0]
    for n in range(1, NB):
        m = jnp.maximum(m, logits[n])
    es = [jnp.exp(l - m) for l in logits]
    s = es[0]
    for n in range(1, NB):
        s = s + es[n]
    inv = 1.0 / s
    h8t = h8t_ref[...]
    acc = jnp.zeros((BC, DP), jnp.float32)
    for n in range(NB):
        w = es[n] * inv
        vf = jax.lax.bitcast_convert_type(gs[n] & _MASK_HI, jnp.float32)
        acc = acc + (jnp.dot(w, h8t, preferred_element_type=jnp.float32)
                     * vf)

    tp = jnp.dot(tgt_ref[...], p_ref[...], preferred_element_type=jnp.float32)
    x = tp + jnp.dot(acc, wo_ref[...], preferred_element_type=jnp.float32)
    mm5 = ma_ref[...]
    mb5 = mb5_ref[...]
    mb = jnp.dot(jnp.dot(x, mm5, preferred_element_type=jnp.float32), mb5,
                 preferred_element_type=jnp.float32)
    xc = x - mb
    vb = jnp.dot(jnp.dot(xc * xc, mm5, preferred_element_type=jnp.float32),
                 mb5, preferred_element_type=jnp.float32)
    xn = xc * lax.rsqrt(vb + 1e-5) * g1_ref[...] + be1_ref[...]
    h = jnp.maximum(
        jnp.dot(xn.astype(jnp.bfloat16), w1_ref[...],
                preferred_element_type=jnp.float32)
        + b1_ref[...], 0.0)
    x2 = xn + (jnp.dot(h.astype(jnp.bfloat16), w2_ref[...],
                       preferred_element_type=jnp.float32)
               + b2_ref[...])
    mb2 = jnp.dot(jnp.dot(x2, mm5, preferred_element_type=jnp.float32), mb5,
                  preferred_element_type=jnp.float32)
    xc2 = x2 - mb2
    vb2 = jnp.dot(jnp.dot(xc2 * xc2, mm5, preferred_element_type=jnp.float32),
                  mb5, preferred_element_type=jnp.float32)
    out_ref[...] = (jnp.dot(xc2 * lax.rsqrt(vb2 + 1e-5), g2p_ref[...],
                            preferred_element_type=jnp.float32)
                    + be2d_ref[...])


def _attn_tail(c0, tgt, kvp, g3, dist_t, p_mat, h8, h8t, wo, mm5, mb5,
               w1k, b1k, w2k, b2k, g1t, be1t, g2p, be2d):
    grid = (NC // BC,)
    nblk = NC // BC
    full = lambda r, c: pl.BlockSpec((r, c), lambda i: (0, 0))
    return pl.pallas_call(
        _attn_body,
        grid=grid,
        in_specs=[
            pl.BlockSpec((BC, DA), lambda i: (c0 * nblk + i, 0)),
            pl.BlockSpec((BC, DP), lambda i: (c0 * nblk + i, 0)),
            pl.BlockSpec((NB, BC, DP), lambda i: (0, i, 0)),
            pl.BlockSpec((NB, BC, NHEAD),
                         lambda i: (0, c0 * nblk + i, 0)),
            full(DA, DA),
            full(DP, NHEAD),
            full(NHEAD, DP),
            full(DP, DA),
            full(DA, AN),
            full(AN, DA),
            full(DA, AN * DFF),
            full(1, AN * DFF),
            full(AN * DFF, DA),
            full(1, DA),
            full(1, DA),
            full(1, DA),
            full(DA, DA),
            full(1, DA),
        ],
        out_specs=pl.BlockSpec((BC, DA), lambda i: (i, 0)),
        out_shape=jax.ShapeDtypeStruct((NC, DA), jnp.float32),
    )(tgt, kvp, g3, dist_t, p_mat, h8, h8t, wo, mm5, mb5, w1k, b1k, w2k,
      b2k, g1t, be1t, g2p, be2d)


def kernel(tgt, index_pair, cnt, sh, dist_atten, Wq, Wk, Wout, W1, b1, W2,
           b2, g1, be1, g2, be2):
    del cnt, sh
    f32 = jnp.float32
    eye5 = jnp.eye(AN, dtype=f32)

    def expand_in(w):
        m = w.T[:, None, None, :] * eye5[None, :, :, None]
        return m.reshape(DA, DA)

    pad = jnp.zeros((DA, DP - DA), f32)
    wkp = jnp.concatenate([expand_in(Wk), pad], axis=1)
    wqp = jnp.concatenate([expand_in(Wq), pad], axis=1)
    p_mat = expand_in(jnp.eye(D_MODEL, dtype=f32))
    wo = jnp.kron(eye5, Wout.T)
    wo384 = jnp.concatenate([wo, jnp.zeros((DP - DA, DA), f32)], axis=0)
    mm5 = jnp.kron(eye5, jnp.full((D_MODEL, 1), 1.0 / D_MODEL, f32))
    mb5 = jnp.kron(eye5, jnp.ones((1, D_MODEL), f32))
    w1k = jnp.kron(eye5, W1.T).astype(jnp.bfloat16)
    w2k = jnp.kron(eye5, W2.T).astype(jnp.bfloat16)
    b1k = jnp.tile(b1, AN)[None, :]
    b2k = jnp.tile(b2, AN)[None, :]
    g1t = jnp.tile(g1, AN)[None, :]
    be1t = jnp.tile(be1, AN)[None, :]
    g2t = jnp.tile(g2, AN)
    be2t = jnp.tile(be2, AN)
    dmaj = jnp.tile(jnp.arange(D_MODEL), AN)
    h8 = (dmaj[:, None] // DPH == jnp.arange(NHEAD)[None, :]).astype(f32)
    h8pad = jnp.concatenate([h8, jnp.zeros((DP - DA, NHEAD), f32)], axis=0)
    h8t = jnp.concatenate([h8.T, jnp.zeros((NHEAD, DP - DA), f32)], axis=1)
    g2p = g2t[:, None] * p_mat.T
    be2d = be2t[None, :] @ p_mat.T

    kvp = _project(tgt, wkp, wqp)
    idx_full = jnp.transpose(
        index_pair.astype(jnp.int32).reshape(NCHUNK, NC, NB),
        (0, 2, 1)).reshape(-1)
    dist_t = jnp.transpose(dist_atten, (1, 0, 2))
    outs = []
    for c in range(NCHUNK):
        g3 = _gather(kvp, idx_full, c).reshape(NB, NC, DP)
        outs.append(_attn_tail(c, tgt, kvp, g3, dist_t, p_mat, h8pad, h8t,
                               wo384, mm5, mb5, w1k, b1k, w2k, b2k, g1t,
                               be1t, g2p, be2d))
    return jnp.concatenate(outs, axis=0)


# speedup vs baseline: 1.0504x; 1.0028x over previous
import functools

import jax
import jax.numpy as jnp
from jax import lax
from jax.experimental import pallas as pl
from jax.experimental.pallas import tpu as pltpu
from jax.experimental.pallas import tpu_sc as plsc

AN = 5
D_MODEL = 64
NHEAD = 8
DPH = D_MODEL // NHEAD
DFF = 256
N = 10000
NB = 16
DA = D_MODEL * AN
DP = 384

CHUNKS = (3200, 3200, 3600)

NWORK = 32
WIN = 144

BA = 1000
BC = 400

_MASK_HI = -65536


def _to_bf16_bits(x):
    b = jax.lax.bitcast_convert_type(x, jnp.int32)
    return ((b + 0x7FFF + ((b >> 16) & 1)) >> 16) & 0xFFFF


def _proj_body(tgt_ref, wk_ref, wq_ref, kv_ref):
    t = tgt_ref[...]
    kf = jnp.dot(t, wk_ref[...], preferred_element_type=jnp.float32)
    vf = jnp.dot(t, wq_ref[...], preferred_element_type=jnp.float32)
    kv_ref[...] = (_to_bf16_bits(vf) << 16) | _to_bf16_bits(kf)


def _project(tgt, wkp, wqp):
    grid = (N // BA,)
    return pl.pallas_call(
        _proj_body,
        grid=grid,
        in_specs=[
            pl.BlockSpec((BA, DA), lambda i: (i, 0)),
            pl.BlockSpec((DA, DP), lambda i: (0, 0)),
            pl.BlockSpec((DA, DP), lambda i: (0, 0)),
        ],
        out_specs=pl.BlockSpec((BA, DP), lambda i: (i, 0)),
        out_shape=jax.ShapeDtypeStruct((N, DP), jnp.int32),
    )(tgt, wkp, wqp)


def _gather(kv, idx_t, pair0, nc):
    mesh = plsc.VectorSubcoreMesh(core_axis_name="c", subcore_axis_name="s")
    pairs_c = nc * NB
    per_w = pairs_c // NWORK
    nwin = -(-per_w // WIN)
    last_off = per_w - WIN

    @functools.partial(
        pl.kernel,
        out_type=jax.ShapeDtypeStruct((pairs_c, DP), jnp.int32),
        mesh=mesh,
        scratch_types=[
            pltpu.VMEM((per_w,), jnp.int32),
            pltpu.VMEM((WIN, DP), jnp.int32),
            pltpu.VMEM((WIN, DP), jnp.int32),
            pltpu.SemaphoreType.DMA,
            pltpu.SemaphoreType.DMA,
            pltpu.SemaphoreType.DMA,
            pltpu.SemaphoreType.DMA,
        ],
    )
    def k(kv_hbm, idx_hbm, g_hbm, idx_v, b0, b1, sg0, sg1, sw0, sw1):
        wid = lax.axis_index("s") * 2 + lax.axis_index("c")
        base = pl.multiple_of(wid * per_w, 8)
        pltpu.sync_copy(idx_hbm.at[pl.ds(pair0 + base, per_w)], idx_v)

        bufs = (b0, b1)
        gsem = (sg0, sg1)
        wsem = (sw0, sw1)

        def wait_gather(par):
            pltpu.make_async_copy(
                kv_hbm.at[pl.ds(0, WIN)], bufs[par], gsem[par]).wait()

        def wait_write(par):
            pltpu.make_async_copy(
                bufs[par], g_hbm.at[pl.ds(0, WIN)], wsem[par]).wait()

        pltpu.async_copy(kv_hbm.at[idx_v.at[pl.ds(0, WIN)]], b0, sg0)

        @pl.loop(0, nwin)
        def _(w):
            for par in (0, 1):
                oth = 1 - par

                @pl.when(lax.rem(w, 2) == par)
                def _(par=par, oth=oth):
                    @pl.when(w >= 1)
                    def _():
                        wait_write(oth)

                    @pl.when(w + 1 < nwin)
                    def _():
                        off_n = pl.multiple_of(
                            jnp.minimum(w * WIN + WIN, last_off), 8)
                        pltpu.async_copy(
                            kv_hbm.at[idx_v.at[pl.ds(off_n, WIN)]],
                            bufs[oth], gsem[oth])

                    wait_gather(par)
                    off = pl.multiple_of(jnp.minimum(w * WIN, last_off), 8)
                    pltpu.async_copy(
                        bufs[par], g_hbm.at[pl.ds(base + off, WIN)],
                        wsem[par])

        wait_write((nwin - 1) % 2)

    return k(kv, idx_t)


def _attn_body(tgt_ref, own_ref, g_ref, dist_ref, p_ref, h8_ref,
               h8t_ref, wo_ref, ma_ref, mb5_ref, w1_ref, b1_ref, w2_ref,
               b2_ref, g1_ref, be1_ref, g2p_ref, be2d_ref, out_ref):
    scale = 1.0 / (DPH ** 0.5)
    qv = jax.lax.bitcast_convert_type(own_ref[...] & _MASK_HI, jnp.float32)
    h8 = h8_ref[...]
    gs = [g_ref[n] for n in range(NB)]
    logits = []
    for n in range(NB):
        kf = jax.lax.bitcast_convert_type(gs[n] << 16, jnp.float32)
        dn = jnp.dot(qv * kf, h8, preferred_element_type=jnp.float32)
        logits.append((dn + dist_ref[n]) * scale)
    m = logits[0]
    for n in range(1, NB):
        m = jnp.maximum(m, logits[n])
    es = [jnp.exp(l - m) for l in logits]
    s = es[0]
    for n in range(1, NB):
        s = s + es[n]
    inv = 1.0 / s
    h8t = h8t_ref[...]
    acc = jnp.zeros((BC, DP), jnp.float32)
    for n in range(NB):
        w = es[n] * inv
        vf = jax.lax.bitcast_convert_type(gs[n] & _MASK_HI, jnp.float32)
        acc = acc + (jnp.dot(w, h8t, preferred_element_type=jnp.float32)
                     * vf)

    tp = jnp.dot(tgt_ref[...], p_ref[...], preferred_element_type=jnp.float32)
    x = tp + jnp.dot(acc, wo_ref[...], preferred_element_type=jnp.float32)
    mm5 = ma_ref[...]
    mb5 = mb5_ref[...]
    mb = jnp.dot(jnp.dot(x, mm5, preferred_element_type=jnp.float32), mb5,
                 preferred_element_type=jnp.float32)
    xc = x - mb
    vb = jnp.dot(jnp.dot(xc * xc, mm5, preferred_element_type=jnp.float32),
                 mb5, preferred_element_type=jnp.float32)
    xn = xc * lax.rsqrt(vb + 1e-5) * g1_ref[...] + be1_ref[...]
    h = jnp.maximum(
        jnp.dot(xn.astype(jnp.bfloat16), w1_ref[...],
                preferred_element_type=jnp.float32)
        + b1_ref[...], 0.0)
    x2 = xn + (jnp.dot(h.astype(jnp.bfloat16), w2_ref[...],
                       preferred_element_type=jnp.float32)
               + b2_ref[...])
    mb2 = jnp.dot(jnp.dot(x2, mm5, preferred_element_type=jnp.float32), mb5,
                  preferred_element_type=jnp.float32)
    xc2 = x2 - mb2
    vb2 = jnp.dot(jnp.dot(xc2 * xc2, mm5, preferred_element_type=jnp.float32),
                  mb5, preferred_element_type=jnp.float32)
    out_ref[...] = (jnp.dot(xc2 * lax.rsqrt(vb2 + 1e-5), g2p_ref[...],
                            preferred_element_type=jnp.float32)
                    + be2d_ref[...])


def _attn_tail(n0, nc, tgt, kvp, g3, dist_t, p_mat, h8, h8t, wo, mm5, mb5,
               w1k, b1k, w2k, b2k, g1t, be1t, g2p, be2d):
    grid = (nc // BC,)
    blk0 = n0 // BC
    full = lambda r, c: pl.BlockSpec((r, c), lambda i: (0, 0))
    return pl.pallas_call(
        _attn_body,
        grid=grid,
        in_specs=[
            pl.BlockSpec((BC, DA), lambda i: (blk0 + i, 0)),
            pl.BlockSpec((BC, DP), lambda i: (blk0 + i, 0)),
            pl.BlockSpec((NB, BC, DP), lambda i: (0, i, 0)),
            pl.BlockSpec((NB, BC, NHEAD),
                         lambda i: (0, blk0 + i, 0)),
            full(DA, DA),
            full(DP, NHEAD),
            full(NHEAD, DP),
            full(DP, DA),
            full(DA, AN),
            full(AN, DA),
            full(DA, AN * DFF),
            full(1, AN * DFF),
            full(AN * DFF, DA),
            full(1, DA),
            full(1, DA),
            full(1, DA),
            full(DA, DA),
            full(1, DA),
        ],
        out_specs=pl.BlockSpec((BC, DA), lambda i: (i, 0)),
        out_shape=jax.ShapeDtypeStruct((nc, DA), jnp.float32),
    )(tgt, kvp, g3, dist_t, p_mat, h8, h8t, wo, mm5, mb5, w1k, b1k, w2k,
      b2k, g1t, be1t, g2p, be2d)


def kernel(tgt, index_pair, cnt, sh, dist_atten, Wq, Wk, Wout, W1, b1, W2,
           b2, g1, be1, g2, be2):
    del cnt, sh
    f32 = jnp.float32
    eye5 = jnp.eye(AN, dtype=f32)

    def expand_in(w):
        m = w.T[:, None, None, :] * eye5[None, :, :, None]
        return m.reshape(DA, DA)

    pad = jnp.zeros((DA, DP - DA), f32)
    wkp = jnp.concatenate([expand_in(Wk), pad], axis=1)
    wqp = jnp.concatenate([expand_in(Wq), pad], axis=1)
    p_mat = expand_in(jnp.eye(D_MODEL, dtype=f32))
    wo = jnp.kron(eye5, Wout.T)
    wo384 = jnp.concatenate([wo, jnp.zeros((DP - DA, DA), f32)], axis=0)
    mm5 = jnp.kron(eye5, jnp.full((D_MODEL, 1), 1.0 / D_MODEL, f32))
    mb5 = jnp.kron(eye5, jnp.ones((1, D_MODEL), f32))
    w1k = jnp.kron(eye5, W1.T).astype(jnp.bfloat16)
    w2k = jnp.kron(eye5, W2.T).astype(jnp.bfloat16)
    b1k = jnp.tile(b1, AN)[None, :]
    b2k = jnp.tile(b2, AN)[None, :]
    g1t = jnp.tile(g1, AN)[None, :]
    be1t = jnp.tile(be1, AN)[None, :]
    g2t = jnp.tile(g2, AN)
    be2t = jnp.tile(be2, AN)
    dmaj = jnp.tile(jnp.arange(D_MODEL), AN)
    h8 = (dmaj[:, None] // DPH == jnp.arange(NHEAD)[None, :]).astype(f32)
    h8pad = jnp.concatenate([h8, jnp.zeros((DP - DA, NHEAD), f32)], axis=0)
    h8t = jnp.concatenate([h8.T, jnp.zeros((NHEAD, DP - DA), f32)], axis=1)
    g2p = g2t[:, None] * p_mat.T
    be2d = be2t[None, :] @ p_mat.T

    kvp = _project(tgt, wkp, wqp)
    idx32 = index_pair.astype(jnp.int32)
    idx_parts = []
    n0 = 0
    for nc in CHUNKS:
        idx_parts.append(
            jnp.transpose(idx32[n0:n0 + nc]).reshape(-1))
        n0 += nc
    idx_full = jnp.concatenate(idx_parts)
    dist_t = jnp.transpose(dist_atten, (1, 0, 2))
    outs = []
    n0 = 0
    for nc in CHUNKS:
        g3 = _gather(kvp, idx_full, n0 * NB, nc).reshape(NB, nc, DP)
        outs.append(_attn_tail(n0, nc, tgt, kvp, g3, dist_t, p_mat, h8pad,
                               h8t, wo384, mm5, mb5, w1k, b1k, w2k, b2k,
                               g1t, be1t, g2p, be2d))
        n0 += nc
    return jnp.concatenate(outs, axis=0)


# speedup vs baseline: 1.0702x; 1.0189x over previous
import functools

import jax
import jax.numpy as jnp
from jax import lax
from jax.experimental import pallas as pl
from jax.experimental.pallas import tpu as pltpu
from jax.experimental.pallas import tpu_sc as plsc

AN = 5
D_MODEL = 64
NHEAD = 8
DPH = D_MODEL // NHEAD
DFF = 256
N = 10000
NB = 16
DA = D_MODEL * AN
DP = 384

CHUNKS = (3200, 3200, 3600)

NWORK = 32
WIN = 40

BA = 1000
BC = 400

_MASK_HI = -65536


def _to_bf16_bits(x):
    b = jax.lax.bitcast_convert_type(x, jnp.int32)
    return ((b + 0x7FFF + ((b >> 16) & 1)) >> 16) & 0xFFFF


def _proj_body(tgt_ref, wk_ref, wq_ref, kv_ref):
    t = tgt_ref[...]
    kf = jnp.dot(t, wk_ref[...], preferred_element_type=jnp.float32)
    vf = jnp.dot(t, wq_ref[...], preferred_element_type=jnp.float32)
    kv_ref[...] = (_to_bf16_bits(vf) << 16) | _to_bf16_bits(kf)


def _project(tgt, wkp, wqp):
    grid = (N // BA,)
    return pl.pallas_call(
        _proj_body,
        grid=grid,
        in_specs=[
            pl.BlockSpec((BA, DA), lambda i: (i, 0)),
            pl.BlockSpec((DA, DP), lambda i: (0, 0)),
            pl.BlockSpec((DA, DP), lambda i: (0, 0)),
        ],
        out_specs=pl.BlockSpec((BA, DP), lambda i: (i, 0)),
        out_shape=jax.ShapeDtypeStruct((N, DP), jnp.int32),
    )(tgt, wkp, wqp)


def _gather(kv, idx_t, pair0, nc):
    mesh = plsc.VectorSubcoreMesh(core_axis_name="c", subcore_axis_name="s")
    pairs_c = nc * NB
    per_w = pairs_c // NWORK
    nwin = -(-per_w // WIN)
    last_off = per_w - WIN

    @functools.partial(
        pl.kernel,
        out_type=jax.ShapeDtypeStruct((pairs_c, DP), jnp.int32),
        mesh=mesh,
        scratch_types=[
            pltpu.VMEM((per_w,), jnp.int32),
            pltpu.VMEM((WIN, DP), jnp.int32),
            pltpu.VMEM((WIN, DP), jnp.int32),
            pltpu.SemaphoreType.DMA,
            pltpu.SemaphoreType.DMA,
            pltpu.SemaphoreType.DMA,
            pltpu.SemaphoreType.DMA,
        ],
    )
    def k(kv_hbm, idx_hbm, g_hbm, idx_v, b0, b1, sg0, sg1, sw0, sw1):
        wid = lax.axis_index("s") * 2 + lax.axis_index("c")
        base = pl.multiple_of(wid * per_w, 8)
        pltpu.sync_copy(idx_hbm.at[pl.ds(pair0 + base, per_w)], idx_v)

        bufs = (b0, b1)
        gsem = (sg0, sg1)
        wsem = (sw0, sw1)

        def wait_gather(par):
            pltpu.make_async_copy(
                kv_hbm.at[pl.ds(0, WIN)], bufs[par], gsem[par]).wait()

        def wait_write(par):
            pltpu.make_async_copy(
                bufs[par], g_hbm.at[pl.ds(0, WIN)], wsem[par]).wait()

        pltpu.async_copy(kv_hbm.at[idx_v.at[pl.ds(0, WIN)]], b0, sg0)

        @pl.loop(0, nwin)
        def _(w):
            for par in (0, 1):
                oth = 1 - par

                @pl.when(lax.rem(w, 2) == par)
                def _(par=par, oth=oth):
                    @pl.when(w >= 1)
                    def _():
                        wait_write(oth)

                    @pl.when(w + 1 < nwin)
                    def _():
                        off_n = pl.multiple_of(
                            jnp.minimum(w * WIN + WIN, last_off), 8)
                        pltpu.async_copy(
                            kv_hbm.at[idx_v.at[pl.ds(off_n, WIN)]],
                            bufs[oth], gsem[oth])

                    wait_gather(par)
                    off = pl.multiple_of(jnp.minimum(w * WIN, last_off), 8)
                    pltpu.async_copy(
                        bufs[par], g_hbm.at[pl.ds(base + off, WIN)],
                        wsem[par])

        wait_write((nwin - 1) % 2)

    return k(kv, idx_t)


def _attn_body(tgt_ref, own_ref, g_ref, dist_ref, p_ref, h8_ref,
               h8t_ref, wo_ref, ma_ref, mb5_ref, w1_ref, b1_ref, w2_ref,
               b2_ref, g1_ref, be1_ref, g2p_ref, be2d_ref, out_ref):
    scale = 1.0 / (DPH ** 0.5)
    qv = jax.lax.bitcast_convert_type(own_ref[...] & _MASK_HI, jnp.float32)
    h8 = h8_ref[...]
    gs = [g_ref[n] for n in range(NB)]
    logits = []
    for n in range(NB):
        kf = jax.lax.bitcast_convert_type(gs[n] << 16, jnp.float32)
        dn = jnp.dot(qv * kf, h8, preferred_element_type=jnp.float32)
        logits.append((dn + dist_ref[n]) * scale)
    m = logits[0]
    for n in range(1, NB):
        m = jnp.maximum(m, logits[n])
    es = [jnp.exp(l - m) for l in logits]
    s = es[0]
    for n in range(1, NB):
        s = s + es[n]
    inv = 1.0 / s
    h8t = h8t_ref[...]
    acc = jnp.zeros((BC, DP), jnp.float32)
    for n in range(NB):
        w = es[n] * inv
        vf = jax.lax.bitcast_convert_type(gs[n] & _MASK_HI, jnp.float32)
        acc = acc + (jnp.dot(w, h8t, preferred_element_type=jnp.float32)
                     * vf)

    tp = jnp.dot(tgt_ref[...], p_ref[...], preferred_element_type=jnp.float32)
    x = tp + jnp.dot(acc, wo_ref[...], preferred_element_type=jnp.float32)
    mm5 = ma_ref[...]
    mb5 = mb5_ref[...]
    mb = jnp.dot(jnp.dot(x, mm5, preferred_element_type=jnp.float32), mb5,
                 preferred_element_type=jnp.float32)
    xc = x - mb
    vb = jnp.dot(jnp.dot(xc * xc, mm5, preferred_element_type=jnp.float32),
                 mb5, preferred_element_type=jnp.float32)
    xn = xc * lax.rsqrt(vb + 1e-5) * g1_ref[...] + be1_ref[...]
    h = jnp.maximum(
        jnp.dot(xn.astype(jnp.bfloat16), w1_ref[...],
                preferred_element_type=jnp.float32)
        + b1_ref[...], 0.0)
    x2 = xn + (jnp.dot(h.astype(jnp.bfloat16), w2_ref[...],
                       preferred_element_type=jnp.float32)
               + b2_ref[...])
    mb2 = jnp.dot(jnp.dot(x2, mm5, preferred_element_type=jnp.float32), mb5,
                  preferred_element_type=jnp.float32)
    xc2 = x2 - mb2
    vb2 = jnp.dot(jnp.dot(xc2 * xc2, mm5, preferred_element_type=jnp.float32),
                  mb5, preferred_element_type=jnp.float32)
    out_ref[...] = (jnp.dot(xc2 * lax.rsqrt(vb2 + 1e-5), g2p_ref[...],
                            preferred_element_type=jnp.float32)
                    + be2d_ref[...])


def _attn_tail(n0, nc, tgt, kvp, g3, dist_t, p_mat, h8, h8t, wo, mm5, mb5,
               w1k, b1k, w2k, b2k, g1t, be1t, g2p, be2d):
    grid = (nc // BC,)
    blk0 = n0 // BC
    full = lambda r, c: pl.BlockSpec((r, c), lambda i: (0, 0))
    return pl.pallas_call(
        _attn_body,
        grid=grid,
        in_specs=[
            pl.BlockSpec((BC, DA), lambda i: (blk0 + i, 0)),
            pl.BlockSpec((BC, DP), lambda i: (blk0 + i, 0)),
            pl.BlockSpec((NB, BC, DP), lambda i: (0, i, 0)),
            pl.BlockSpec((NB, BC, NHEAD),
                         lambda i: (0, blk0 + i, 0)),
            full(DA, DA),
            full(DP, NHEAD),
            full(NHEAD, DP),
            full(DP, DA),
            full(DA, AN),
            full(AN, DA),
            full(DA, AN * DFF),
            full(1, AN * DFF),
            full(AN * DFF, DA),
            full(1, DA),
            full(1, DA),
            full(1, DA),
            full(DA, DA),
            full(1, DA),
        ],
        out_specs=pl.BlockSpec((BC, DA), lambda i: (i, 0)),
        out_shape=jax.ShapeDtypeStruct((nc, DA), jnp.float32),
    )(tgt, kvp, g3, dist_t, p_mat, h8, h8t, wo, mm5, mb5, w1k, b1k, w2k,
      b2k, g1t, be1t, g2p, be2d)


def kernel(tgt, index_pair, cnt, sh, dist_atten, Wq, Wk, Wout, W1, b1, W2,
           b2, g1, be1, g2, be2):
    del cnt, sh
    f32 = jnp.float32
    eye5 = jnp.eye(AN, dtype=f32)

    def expand_in(w):
        m = w.T[:, None, None, :] * eye5[None, :, :, None]
        return m.reshape(DA, DA)

    pad = jnp.zeros((DA, DP - DA), f32)
    wkp = jnp.concatenate([expand_in(Wk), pad], axis=1)
    wqp = jnp.concatenate([expand_in(Wq), pad], axis=1)
    p_mat = expand_in(jnp.eye(D_MODEL, dtype=f32))
    wo = jnp.kron(eye5, Wout.T)
    wo384 = jnp.concatenate([wo, jnp.zeros((DP - DA, DA), f32)], axis=0)
    mm5 = jnp.kron(eye5, jnp.full((D_MODEL, 1), 1.0 / D_MODEL, f32))
    mb5 = jnp.kron(eye5, jnp.ones((1, D_MODEL), f32))
    w1k = jnp.kron(eye5, W1.T).astype(jnp.bfloat16)
    w2k = jnp.kron(eye5, W2.T).astype(jnp.bfloat16)
    b1k = jnp.tile(b1, AN)[None, :]
    b2k = jnp.tile(b2, AN)[None, :]
    g1t = jnp.tile(g1, AN)[None, :]
    be1t = jnp.tile(be1, AN)[None, :]
    g2t = jnp.tile(g2, AN)
    be2t = jnp.tile(be2, AN)
    dmaj = jnp.tile(jnp.arange(D_MODEL), AN)
    h8 = (dmaj[:, None] // DPH == jnp.arange(NHEAD)[None, :]).astype(f32)
    h8pad = jnp.concatenate([h8, jnp.zeros((DP - DA, NHEAD), f32)], axis=0)
    h8t = jnp.concatenate([h8.T, jnp.zeros((NHEAD, DP - DA), f32)], axis=1)
    g2p = g2t[:, None] * p_mat.T
    be2d = be2t[None, :] @ p_mat.T

    kvp = _project(tgt, wkp, wqp)
    idx32 = index_pair.astype(jnp.int32)
    idx_parts = []
    n0 = 0
    for nc in CHUNKS:
        idx_parts.append(
            jnp.transpose(idx32[n0:n0 + nc]).reshape(-1))
        n0 += nc
    idx_full = jnp.concatenate(idx_parts)
    dist_t = jnp.transpose(dist_atten, (1, 0, 2))
    outs = []
    n0 = 0
    for nc in CHUNKS:
        g3 = _gather(kvp, idx_full, n0 * NB, nc).reshape(NB, nc, DP)
        outs.append(_attn_tail(n0, nc, tgt, kvp, g3, dist_t, p_mat, h8pad,
                               h8t, wo384, mm5, mb5, w1k, b1k, w2k, b2k,
                               g1t, be1t, g2p, be2d))
        n0 += nc
    return jnp.concatenate(outs, axis=0)
